# baseline (speedup 1.0000x reference)
import functools

import jax
import jax.numpy as jnp
from jax.experimental import pallas as pl

_BATCH = 16384
_EMBED = 64
_TILE = 1024


def _mlp_body(ue_ref, ie_ref, w1u_ref, w1i_ref, b1_ref, w2_ref, b2_ref,
              w3_ref, b3_ref, w4_ref, b4_ref, out_ref):
    dot = functools.partial(jnp.dot, preferred_element_type=jnp.float32)
    bf = jnp.bfloat16
    h = dot(ue_ref[...].astype(bf), w1u_ref[...]) + dot(
        ie_ref[...].astype(bf), w1i_ref[...])
    h = jnp.maximum(h + b1_ref[...], 0.0)
    h = jnp.maximum(dot(h.astype(bf), w2_ref[...]) + b2_ref[...], 0.0)
    h = jnp.maximum(dot(h.astype(bf), w3_ref[...]) + b3_ref[...], 0.0)
    out_ref[...] = jnp.maximum(dot(h.astype(bf), w4_ref[...]) + b4_ref[...], 0.0)


def _mlp(ue, ie, W1u, W1i, b1, W2, b2, W3, b3, W4, b4):
    full = lambda r, c: pl.BlockSpec((r, c), lambda i: (0, 0))
    return pl.pallas_call(
        _mlp_body,
        grid=(_BATCH // _TILE,),
        in_specs=[
            pl.BlockSpec((_TILE, _EMBED), lambda i: (i, 0)),
            pl.BlockSpec((_TILE, _EMBED), lambda i: (i, 0)),
            full(_EMBED, 1024), full(_EMBED, 1024), full(1, 1024),
            full(1024, 512), full(1, 512),
            full(512, 256), full(1, 256),
            full(256, 128), full(1, 128),
        ],
        out_specs=pl.BlockSpec((_TILE, 128), lambda i: (i, 0)),
        out_shape=jax.ShapeDtypeStruct((_BATCH, 128), jnp.float32),
    )(ue, ie, W1u, W1i, b1, W2, b2, W3, b3, W4, b4)


def kernel(user_batch, item_batch, user_table, item_table,
           W1, b1, W2, b2, W3, b3, W4, b4):
    ue = jnp.take(user_table, user_batch, axis=0)
    ie = jnp.take(item_table, item_batch, axis=0)
    bf = jnp.bfloat16
    return _mlp(ue, ie, W1[:_EMBED].astype(bf), W1[_EMBED:].astype(bf),
                b1.reshape(1, -1), W2.astype(bf), b2.reshape(1, -1),
                W3.astype(bf), b3.reshape(1, -1), W4.astype(bf),
                b4.reshape(1, -1))
